# Initial kernel scaffold; baseline (speedup 1.0000x reference)
import jax
import jax.numpy as jnp
from jax.experimental import pallas as pl


def kernel(pred_heatmap, pred_regression):
    raise NotImplementedError("write your pallas kernel here")


# speedup vs baseline: 2.8480x; 2.8480x over previous
import functools

import jax
import jax.numpy as jnp
from jax.experimental import pallas as pl
from jax.experimental.pallas import tpu as pltpu

B, C, R, H, W = 4, 3, 50, 192, 640
HW = H * W
K = 100
KPAD = 128
DOWN = 4
DET_TH = 0.2
FX = 721.5377
FY = 721.5377
CX = W * DOWN / 2.0
CY = H * DOWN / 2.0
PI = 3.14159265358979323846
NEG = -2.0
BIGI = 2 ** 30


def _topk_map_kernel(heat_ref, out_s_ref, out_i_ref, scr):
    x = heat_ref[0]
    ninf = jnp.float32(-jnp.inf)
    lcol = jnp.concatenate([x[:, 1:], jnp.full((H, 1), ninf, jnp.float32)], axis=1)
    rcol = jnp.concatenate([jnp.full((H, 1), ninf, jnp.float32), x[:, :-1]], axis=1)
    cm = jnp.maximum(jnp.maximum(lcol, rcol), x)
    urow = jnp.concatenate([cm[1:], jnp.full((1, W), ninf, jnp.float32)], axis=0)
    drow = jnp.concatenate([jnp.full((1, W), ninf, jnp.float32), cm[:-1]], axis=0)
    hmax = jnp.maximum(jnp.maximum(urow, drow), cm)
    v = jnp.where(hmax == x, x, 0.0)

    scr[...] = v
    rmax0 = jnp.max(v, axis=1, keepdims=True)
    riota = jax.lax.broadcasted_iota(jnp.int32, (H, 1), 0)
    ciota = jax.lax.broadcasted_iota(jnp.int32, (1, W), 1)
    kiota = jax.lax.broadcasted_iota(jnp.int32, (1, KPAD), 1)

    def body(i, carry):
        rmax, sc, ia = carry
        m = jnp.max(rmax)
        r = jnp.min(jnp.where(rmax == m, riota, BIGI))
        row = scr[pl.ds(r, 1), :]
        c = jnp.min(jnp.where(row == m, ciota, BIGI))
        flat = r * W + c
        newrow = jnp.where(ciota == c, NEG, row)
        scr[pl.ds(r, 1), :] = newrow
        rmax = jnp.where(riota == r, jnp.max(newrow), rmax)
        sc = jnp.where(kiota == i, m, sc)
        ia = jnp.where(kiota == i, flat, ia)
        return rmax, sc, ia

    sc0 = jnp.full((1, KPAD), NEG, jnp.float32)
    ia0 = jnp.zeros((1, KPAD), jnp.int32)
    _, sc, ia = jax.lax.fori_loop(0, K, body, (rmax0, sc0, ia0))
    out_s_ref[0] = sc
    out_i_ref[0] = ia


def _merge_math_kernel(sc_ref, ind_ref, feat_ref, out_ref):
    w0 = sc_ref[0]
    inds_v = ind_ref[0]
    piota = jax.lax.broadcasted_iota(jnp.int32, (1, C * KPAD), 1)
    kiota = jax.lax.broadcasted_iota(jnp.int32, (1, KPAD), 1)
    pkiota = jax.lax.broadcasted_iota(jnp.int32, (R, KPAD), 1)
    liota = jax.lax.broadcasted_iota(jnp.int32, (R, 1, 128), 2)

    def body(i, carry):
        w, sc, ia, ca, pois = carry
        m = jnp.max(w)
        p = jnp.min(jnp.where(w == m, piota, BIGI))
        ind = jnp.sum(jnp.where(piota == p, inds_v, 0))
        cls = p // KPAD
        sub = ind // 128
        lane = ind % 128
        blk = feat_ref[0, :, pl.ds(sub, 1), :]
        col = jnp.sum(jnp.where(liota == lane, blk, 0.0), axis=2)
        pois = jnp.where(pkiota == i, col, pois)
        w = jnp.where(piota == p, NEG, w)
        sc = jnp.where(kiota == i, m, sc)
        ia = jnp.where(kiota == i, ind, ia)
        ca = jnp.where(kiota == i, cls, ca)
        return w, sc, ia, ca, pois

    sc0 = jnp.full((1, KPAD), NEG, jnp.float32)
    ia0 = jnp.zeros((1, KPAD), jnp.int32)
    ca0 = jnp.zeros((1, KPAD), jnp.int32)
    pois0 = jnp.zeros((R, KPAD), jnp.float32)
    _, scores, ia, ca, pois = jax.lax.fori_loop(
        0, K, body, (w0, sc0, ia0, ca0, pois0))

    f32 = jnp.float32
    xs = (ia % W).astype(f32)
    ys = (ia // W).astype(f32)

    def ch(j):
        return pois[j:j + 1, :]

    relu = lambda t: jnp.maximum(t, 0.0)
    x1 = (xs - relu(ch(0))) * DOWN
    y1 = (ys - relu(ch(1))) * DOWN
    x2 = (xs + relu(ch(2))) * DOWN
    y2 = (ys + relu(ch(3))) * DOWN
    xhi = f32(W * DOWN - 1.0)
    yhi = f32(H * DOWN - 1.0)
    x1 = jnp.clip(x1, 0.0, xhi)
    x2 = jnp.clip(x2, 0.0, xhi)
    y1 = jnp.clip(y1, 0.0, yhi)
    y2 = jnp.clip(y2, 0.0, yhi)

    caf = ca
    def sel3(a, b, c):
        return jnp.where(caf == 0, f32(a), jnp.where(caf == 1, f32(b), f32(c)))
    dim0 = sel3(3.88, 0.84, 1.76) * jnp.exp(ch(6))
    dim1 = sel3(1.53, 1.76, 1.74) * jnp.exp(ch(7))
    dim2 = sel3(1.63, 0.66, 0.60) * jnp.exp(ch(8))

    sig = 1.0 / (1.0 + jnp.exp(-ch(25)))
    depth = jnp.clip(1.0 / (sig + 1e-6) - 1.0, 0.1, 100.0)
    projx = (xs + ch(4)) * DOWN
    projy = (ys + ch(5)) * DOWN
    locx = (projx - CX) * depth / FX
    locy = (projy - CY) * depth / FY + dim1 / 2.0

    d0 = ch(10) - ch(9)
    d1 = ch(12) - ch(11)
    d2 = ch(14) - ch(13)
    d3 = ch(16) - ch(15)
    best = d0
    bidx = jnp.zeros_like(caf)
    for j, dj in ((1, d1), (2, d2), (3, d3)):
        take = dj > best
        bidx = jnp.where(take, j, bidx)
        best = jnp.where(take, dj, best)
    def selbin(o0, o1, o2, o3):
        return jnp.where(bidx == 0, o0, jnp.where(bidx == 1, o1,
                         jnp.where(bidx == 2, o2, o3)))
    sel_s = selbin(ch(17), ch(19), ch(21), ch(23))
    sel_c = selbin(ch(18), ch(20), ch(22), ch(24))
    bctr = selbin(jnp.full_like(best, 0.0), jnp.full_like(best, 0.5 * PI),
                  jnp.full_like(best, PI), jnp.full_like(best, -0.5 * PI))

    def wrap(a):
        t = a + PI
        return t - jnp.floor(t / (2.0 * PI)) * (2.0 * PI) - PI

    alpha = wrap(jnp.arctan2(sel_s, sel_c) + bctr)
    roty = wrap(alpha + jnp.arctan2(locx, depth))
    conf = 1.0 - jnp.clip(jnp.exp(ch(26)), 0.01, 1.0)
    fsc = scores * conf

    valid = scores >= DET_TH
    rows = (caf.astype(f32), alpha, x1, y1, x2, y2, dim0, dim1, dim2,
            locx, locy, depth, roty, fsc)
    for j, val in enumerate(rows):
        out_ref[0, j:j + 1, :] = jnp.where(valid, val, 0.0)
    zero = jnp.zeros((1, 1, KPAD), jnp.float32)
    out_ref[0, 14:15, :] = zero[0]
    out_ref[0, 15:16, :] = zero[0]


def kernel(pred_heatmap, pred_regression):
    heat = pred_heatmap.reshape(B * C, H, W)
    sc_a, ind_a = pl.pallas_call(
        _topk_map_kernel,
        grid=(B * C,),
        in_specs=[pl.BlockSpec((1, H, W), lambda i: (i, 0, 0))],
        out_specs=[pl.BlockSpec((1, 1, KPAD), lambda i: (i, 0, 0)),
                   pl.BlockSpec((1, 1, KPAD), lambda i: (i, 0, 0))],
        out_shape=[jax.ShapeDtypeStruct((B * C, 1, KPAD), jnp.float32),
                   jax.ShapeDtypeStruct((B * C, 1, KPAD), jnp.int32)],
        scratch_shapes=[pltpu.VMEM((H, W), jnp.float32)],
    )(heat)

    sc_b = sc_a.reshape(B, 1, C * KPAD)
    ind_b = ind_a.reshape(B, 1, C * KPAD)
    feat = pred_regression.reshape(B, R, HW // 128, 128)
    out = pl.pallas_call(
        _merge_math_kernel,
        grid=(B,),
        in_specs=[pl.BlockSpec((1, 1, C * KPAD), lambda i: (i, 0, 0)),
                  pl.BlockSpec((1, 1, C * KPAD), lambda i: (i, 0, 0)),
                  pl.BlockSpec((1, R, HW // 128, 128), lambda i: (i, 0, 0, 0))],
        out_specs=pl.BlockSpec((1, 16, KPAD), lambda i: (i, 0, 0)),
        out_shape=jax.ShapeDtypeStruct((B, 16, KPAD), jnp.float32),
    )(sc_b, ind_b, feat)

    return out.transpose(0, 2, 1)[:, :K, :14].reshape(B * K, 14)


# speedup vs baseline: 5.1496x; 1.8081x over previous
import jax
import jax.numpy as jnp
from jax.experimental import pallas as pl
from jax.experimental.pallas import tpu as pltpu

B, C, R, H, W = 4, 3, 50, 192, 640
HW = H * W
CH = C * H
K = 100
KPAD = 128
DOWN = 4
DET_TH = 0.2
FX = 721.5377
FY = 721.5377
CX = W * DOWN / 2.0
CY = H * DOWN / 2.0
PI = 3.14159265358979323846
NEG = -2.0
BIGI = 2 ** 30


def _topk_kernel(heat_ref, out_s_ref, out_i_ref, scr):
    ninf = jnp.float32(-jnp.inf)
    stats_list = []
    for c in range(C):
        x = heat_ref[0, c]
        lcol = jnp.concatenate(
            [x[:, 1:], jnp.full((H, 1), ninf, jnp.float32)], axis=1)
        rcol = jnp.concatenate(
            [jnp.full((H, 1), ninf, jnp.float32), x[:, :-1]], axis=1)
        cm = jnp.maximum(jnp.maximum(lcol, rcol), x)
        urow = jnp.concatenate(
            [cm[1:], jnp.full((1, W), ninf, jnp.float32)], axis=0)
        drow = jnp.concatenate(
            [jnp.full((1, W), ninf, jnp.float32), cm[:-1]], axis=0)
        hmax = jnp.maximum(jnp.maximum(urow, drow), cm)
        v = jnp.where(hmax == x, x, 0.0)
        scr[pl.ds(c * H, H), :] = v
        stats_list.append(jnp.max(v.reshape(H // 8, 8, W), axis=2))
    stats0 = jnp.concatenate(stats_list, axis=0)

    g8iota = (jax.lax.broadcasted_iota(jnp.int32, (CH // 8, 8), 0) * 8
              + jax.lax.broadcasted_iota(jnp.int32, (CH // 8, 8), 1))
    ciota = jax.lax.broadcasted_iota(jnp.int32, (1, W), 1)
    kiota = jax.lax.broadcasted_iota(jnp.int32, (1, KPAD), 1)

    def body(i, carry):
        stats, sc, ia = carry
        m = jnp.max(stats)
        r = jnp.min(jnp.where(stats == m, g8iota, BIGI))
        row = scr[pl.ds(r, 1), :]
        col = jnp.min(jnp.where(row == m, ciota, BIGI))
        flat = r * W + col
        newrow = jnp.where(ciota == col, NEG, row)
        scr[pl.ds(r, 1), :] = newrow
        stats = jnp.where(g8iota == r, jnp.max(newrow), stats)
        sc = jnp.where(kiota == i, m, sc)
        ia = jnp.where(kiota == i, flat, ia)
        return stats, sc, ia

    sc0 = jnp.full((1, KPAD), NEG, jnp.float32)
    ia0 = jnp.zeros((1, KPAD), jnp.int32)
    _, sc, ia = jax.lax.fori_loop(0, K, body, (stats0, sc0, ia0))
    out_s_ref[0] = sc
    out_i_ref[0] = ia


def _gather_math_kernel(sc_ref, ind_ref, feat_ref, out_ref):
    scores = sc_ref[0]
    ia = ind_ref[0]
    kiota = jax.lax.broadcasted_iota(jnp.int32, (1, KPAD), 1)
    pkiota = jax.lax.broadcasted_iota(jnp.int32, (R, KPAD), 1)
    liota = jax.lax.broadcasted_iota(jnp.int32, (R, 1, 128), 2)

    def body(i, pois):
        ind = jnp.sum(jnp.where(kiota == i, ia, 0))
        hw = ind % HW
        sub = hw // 128
        lane = hw % 128
        blk = feat_ref[0, :, pl.ds(sub, 1), :]
        col = jnp.sum(jnp.where(liota == lane, blk, 0.0), axis=2)
        return jnp.where(pkiota == i, col, pois)

    pois = jax.lax.fori_loop(0, K, body, jnp.zeros((R, KPAD), jnp.float32))

    f32 = jnp.float32
    hwv = ia % HW
    caf = ia // HW
    xs = (hwv % W).astype(f32)
    ys = (hwv // W).astype(f32)

    def ch(j):
        return pois[j:j + 1, :]

    relu = lambda t: jnp.maximum(t, 0.0)
    x1 = (xs - relu(ch(0))) * DOWN
    y1 = (ys - relu(ch(1))) * DOWN
    x2 = (xs + relu(ch(2))) * DOWN
    y2 = (ys + relu(ch(3))) * DOWN
    xhi = f32(W * DOWN - 1.0)
    yhi = f32(H * DOWN - 1.0)
    x1 = jnp.clip(x1, 0.0, xhi)
    x2 = jnp.clip(x2, 0.0, xhi)
    y1 = jnp.clip(y1, 0.0, yhi)
    y2 = jnp.clip(y2, 0.0, yhi)

    def sel3(a, b, c):
        return jnp.where(caf == 0, f32(a), jnp.where(caf == 1, f32(b), f32(c)))
    dim0 = sel3(3.88, 0.84, 1.76) * jnp.exp(ch(6))
    dim1 = sel3(1.53, 1.76, 1.74) * jnp.exp(ch(7))
    dim2 = sel3(1.63, 0.66, 0.60) * jnp.exp(ch(8))

    sig = 1.0 / (1.0 + jnp.exp(-ch(25)))
    depth = jnp.clip(1.0 / (sig + 1e-6) - 1.0, 0.1, 100.0)
    projx = (xs + ch(4)) * DOWN
    projy = (ys + ch(5)) * DOWN
    locx = (projx - CX) * depth / FX
    locy = (projy - CY) * depth / FY + dim1 / 2.0

    d0 = ch(10) - ch(9)
    d1 = ch(12) - ch(11)
    d2 = ch(14) - ch(13)
    d3 = ch(16) - ch(15)
    best = d0
    bidx = jnp.zeros_like(caf)
    for j, dj in ((1, d1), (2, d2), (3, d3)):
        take = dj > best
        bidx = jnp.where(take, j, bidx)
        best = jnp.where(take, dj, best)

    def selbin(o0, o1, o2, o3):
        return jnp.where(bidx == 0, o0, jnp.where(bidx == 1, o1,
                         jnp.where(bidx == 2, o2, o3)))
    sel_s = selbin(ch(17), ch(19), ch(21), ch(23))
    sel_c = selbin(ch(18), ch(20), ch(22), ch(24))
    bctr = selbin(jnp.full_like(best, 0.0), jnp.full_like(best, 0.5 * PI),
                  jnp.full_like(best, PI), jnp.full_like(best, -0.5 * PI))

    def wrap(a):
        t = a + PI
        return t - jnp.floor(t / (2.0 * PI)) * (2.0 * PI) - PI

    alpha = wrap(jnp.arctan2(sel_s, sel_c) + bctr)
    roty = wrap(alpha + jnp.arctan2(locx, depth))
    conf = 1.0 - jnp.clip(jnp.exp(ch(26)), 0.01, 1.0)
    fsc = scores * conf

    valid = scores >= DET_TH
    rows = (caf.astype(f32), alpha, x1, y1, x2, y2, dim0, dim1, dim2,
            locx, locy, depth, roty, fsc)
    for j, val in enumerate(rows):
        out_ref[0, j:j + 1, :] = jnp.where(valid, val, 0.0)
    zero = jnp.zeros((1, KPAD), jnp.float32)
    out_ref[0, 14:15, :] = zero
    out_ref[0, 15:16, :] = zero


def kernel(pred_heatmap, pred_regression):
    sc_a, ind_a = pl.pallas_call(
        _topk_kernel,
        grid=(B,),
        in_specs=[pl.BlockSpec((1, C, H, W), lambda i: (i, 0, 0, 0))],
        out_specs=[pl.BlockSpec((1, 1, KPAD), lambda i: (i, 0, 0)),
                   pl.BlockSpec((1, 1, KPAD), lambda i: (i, 0, 0))],
        out_shape=[jax.ShapeDtypeStruct((B, 1, KPAD), jnp.float32),
                   jax.ShapeDtypeStruct((B, 1, KPAD), jnp.int32)],
        scratch_shapes=[pltpu.VMEM((CH, W), jnp.float32)],
    )(pred_heatmap)

    feat = pred_regression.reshape(B, R, HW // 128, 128)
    out = pl.pallas_call(
        _gather_math_kernel,
        grid=(B,),
        in_specs=[pl.BlockSpec((1, 1, KPAD), lambda i: (i, 0, 0)),
                  pl.BlockSpec((1, 1, KPAD), lambda i: (i, 0, 0)),
                  pl.BlockSpec((1, R, HW // 128, 128), lambda i: (i, 0, 0, 0))],
        out_specs=pl.BlockSpec((1, 16, KPAD), lambda i: (i, 0, 0)),
        out_shape=jax.ShapeDtypeStruct((B, 16, KPAD), jnp.float32),
    )(sc_a, ind_a, feat)

    return out.transpose(0, 2, 1)[:, :K, :14].reshape(B * K, 14)
